# Initial kernel scaffold; baseline (speedup 1.0000x reference)
import jax
import jax.numpy as jnp
from jax.experimental import pallas as pl


def kernel(feat0, feat1, rois, roi_inds):
    raise NotImplementedError("write your pallas kernel here")


# speedup vs baseline: 1.9466x; 1.9466x over previous
import functools

import jax
import jax.numpy as jnp
from jax import lax
from jax.experimental import pallas as pl
from jax.experimental.pallas import tpu as pltpu
from jax.experimental.pallas import tpu_sc as plsc

L = 16
N_ROIS = 512
NC, NS = 2, 16
NW = NC * NS
ROIS_PER_W = N_ROIS // NW
C0, H0 = 192, 64
C1, H1 = 384, 32
OUT_PIX = 49
OFF1 = C0 * OUT_PIX
OUT_COLS = (C0 + C1) * OUT_PIX


def _splat(v):
    return jnp.full((L,), v, jnp.int32)


def _sc_body(t0, t1, rois_h, inds_h, out,
             buf0, buf1, out_t, idxA, idxB, wA, wB, rois_v, inds_v, sem):
    wid = lax.axis_index("s") * NC + lax.axis_index("c")
    base = wid * ROIS_PER_W
    pltpu.sync_copy(rois_h.at[pl.ds(base * 4, 4 * ROIS_PER_W)], rois_v)
    pltpu.sync_copy(inds_h.at[pl.ds(base, ROIS_PER_W)], inds_v)
    iota = lax.iota(jnp.int32, L)
    posc = [iota * OUT_PIX + k * (16 * OUT_PIX) for k in range(C1 // 16)]
    halves = ((0, 25, idxA, wA), (25, 24, idxB, wB))

    def roi_body(r, carry):
        x1 = plsc.load_gather(rois_v, [_splat(4 * r)])
        y1 = plsc.load_gather(rois_v, [_splat(4 * r + 1)])
        x2 = plsc.load_gather(rois_v, [_splat(4 * r + 2)])
        y2 = plsc.load_gather(rois_v, [_splat(4 * r + 3)])
        bI = plsc.load_gather(inds_v, [_splat(r)])
        n = base + r

        for tab, buf, C, HW, inv_s, out_off in (
                (t0, buf0, C0, H0, 0.125, 0),
                (t1, buf1, C1, H1, 0.0625, OFF1)):
            sx1 = x1 * inv_s
            sy1 = y1 * inv_s
            dx = x2 * inv_s - sx1
            dy = y2 * inv_s - sy1
            bbase = bI * (HW * HW)
            fmax = float(HW - 1)
            for p0, npix, idxR, wR in halves:
                for c in range(2):
                    q = iota + c * 16
                    p = jnp.minimum(q + p0, 48)
                    i = lax.shift_right_logical(p * 9363, 16)
                    j = p - i * 7
                    ty = i.astype(jnp.float32) * (1.0 / 6.0)
                    tx = j.astype(jnp.float32) * (1.0 / 6.0)
                    ys = sy1 + dy * ty
                    xs = sx1 + dx * tx
                    ys = jnp.minimum(jnp.maximum(ys, 0.0), fmax)
                    xs = jnp.minimum(jnp.maximum(xs, 0.0), fmax)
                    y0i = ys.astype(jnp.int32)
                    x0i = xs.astype(jnp.int32)
                    wy = ys - y0i.astype(jnp.float32)
                    wx = xs - x0i.astype(jnp.float32)
                    y1i = jnp.minimum(y0i + 1, HW - 1)
                    x1i = jnp.minimum(x0i + 1, HW - 1)
                    ry0 = bbase + y0i * HW
                    ry1 = bbase + y1i * HW
                    msk = q < 25
                    qp = q * 4
                    plsc.store_scatter(idxR, [qp], ry0 + x0i, mask=msk)
                    plsc.store_scatter(idxR, [qp + 1], ry0 + x1i, mask=msk)
                    plsc.store_scatter(idxR, [qp + 2], ry1 + x0i, mask=msk)
                    plsc.store_scatter(idxR, [qp + 3], ry1 + x1i, mask=msk)
                    omy = 1.0 - wy
                    omx = 1.0 - wx
                    sl = pl.ds(c * 16, 16)
                    wR[0, sl] = omy * omx
                    wR[1, sl] = omy * wx
                    wR[2, sl] = wy * omx
                    wR[3, sl] = wy * wx

                pltpu.async_copy(tab.at[idxR], buf, sem).wait()

                def q_body(q, carry2, p0=p0, wR=wR, buf=buf, C=C, out_off=out_off):
                    qs = _splat(q)
                    a = plsc.load_gather(wR, [_splat(0), qs])
                    b = plsc.load_gather(wR, [_splat(1), qs])
                    cc = plsc.load_gather(wR, [_splat(2), qs])
                    d = plsc.load_gather(wR, [_splat(3), qs])
                    p = p0 + q
                    r0 = 4 * q
                    for k in range(C // 16):
                        sl = pl.ds(k * 16, 16)
                        v00 = buf[r0, sl]
                        v01 = buf[r0 + 1, sl]
                        v10 = buf[r0 + 2, sl]
                        v11 = buf[r0 + 3, sl]
                        acc = a * v00 + b * v01 + cc * v10 + d * v11
                        plsc.store_scatter(out_t, [posc[k] + p], acc)
                    return carry2

                lax.fori_loop(0, npix, q_body, 0)

            sz = C * OUT_PIX
            pltpu.sync_copy(out_t.at[pl.ds(0, sz)],
                            out.at[pl.ds(n * OUT_COLS + out_off, sz)])
        return carry

    lax.fori_loop(0, ROIS_PER_W, roi_body, 0)


@functools.partial(
    pl.kernel,
    out_type=jax.ShapeDtypeStruct((N_ROIS * OUT_COLS,), jnp.float32),
    mesh=plsc.VectorSubcoreMesh(core_axis_name="c", subcore_axis_name="s"),
    compiler_params=pltpu.CompilerParams(needs_layout_passes=False,
                                         use_tc_tiling_on_sc=False),
    scratch_types=[
        pltpu.VMEM((100, C0), jnp.float32),
        pltpu.VMEM((100, C1), jnp.float32),
        pltpu.VMEM((C1 * OUT_PIX,), jnp.float32),
        pltpu.VMEM((100,), jnp.int32),
        pltpu.VMEM((100,), jnp.int32),
        pltpu.VMEM((4, 32), jnp.float32),
        pltpu.VMEM((4, 32), jnp.float32),
        pltpu.VMEM((4 * ROIS_PER_W,), jnp.float32),
        pltpu.VMEM((ROIS_PER_W,), jnp.int32),
        pltpu.SemaphoreType.DMA,
    ],
)
def _sc_call(t0, t1, rois_h, inds_h, out, *scratch):
    _sc_body(t0, t1, rois_h, inds_h, out, *scratch)


def kernel(feat0, feat1, rois, roi_inds):
    t0 = jnp.transpose(feat0, (0, 2, 3, 1)).reshape(2 * H0 * H0, C0)
    t1 = jnp.transpose(feat1, (0, 2, 3, 1)).reshape(2 * H1 * H1, C1)
    out = _sc_call(t0, t1, rois.reshape(-1), roi_inds)
    return out.reshape(N_ROIS, C0 + C1, 7, 7)
